# baseline (speedup 1.0000x reference)
import functools

import jax
import jax.numpy as jnp
import numpy as np
from jax.experimental import pallas as pl

_RC = 6.0
_BLOCK_I = 128


def _pair_kernel(pos_blk_ref, pos_t_ref, pos_flat_ref, aux_ref,
                 adj_ref, sft_ref, vec_ref, sod_ref, *, n, block_i, rc2):
    pid = pl.program_id(0)
    row0 = pid * block_i

    rows = row0 + jax.lax.broadcasted_iota(jnp.int32, (block_i, n), 0)
    cols = jax.lax.broadcasted_iota(jnp.int32, (block_i, n), 1)
    sod = jnp.zeros((block_i, n), jnp.float32)
    for c in range(3):
        pj = pos_t_ref[c:c + 1, :]
        pi = pos_blk_ref[:, c:c + 1]
        d = pj - pi
        f = d * aux_ref[2, 2 * c]
        s = -jnp.round(f)
        v = d + s * aux_ref[2, 2 * c + 1]
        sod = sod + v * v
    mask = (sod < rc2) & (rows != cols)
    adj_ref[...] = mask.astype(jnp.int32)
    sod_ref[...] = jnp.where(mask, sod, 0.0)

    pos_i = (pos_blk_ref[:, 0:1] * aux_ref[3:4, :] +
             pos_blk_ref[:, 1:2] * aux_ref[4:5, :] +
             pos_blk_ref[:, 2:3] * aux_ref[5:6, :])
    dv = pos_flat_ref[...] - pos_i
    f = dv * aux_ref[0:1, :]
    s = -jnp.round(f)
    v = dv + s * aux_ref[1:2, :]
    vsq = v * v
    ssum = vsq + jnp.roll(vsq, -1, axis=1) + jnp.roll(vsq, -2, axis=1)
    lane = jax.lax.broadcasted_iota(jnp.int32, (1, 3 * n), 1)
    is0 = (lane % 3) == 0
    sodg = jnp.where(is0, ssum, 0.0)
    sodr = sodg + jnp.roll(sodg, 1, axis=1) + jnp.roll(sodg, 2, axis=1)
    jg = lane // 3
    rows6 = row0 + jax.lax.broadcasted_iota(jnp.int32, (block_i, 1), 0)
    maskr = (sodr < rc2) & (jg != rows6)
    vec_ref[...] = jnp.where(maskr, v, 0.0)
    sft_ref[...] = jnp.where(maskr, s, 0.0).astype(jnp.int32)


def kernel(pos, cel):
    n = pos.shape[0]
    block_i = _BLOCK_I
    grid = n // block_i
    rc2 = np.float32(_RC * _RC)

    inv_cel = jnp.linalg.inv(cel)
    dinv = jnp.diagonal(inv_cel).astype(jnp.float32)
    dcel = jnp.diagonal(cel).astype(jnp.float32)

    pos_t = pos.T
    pos_flat = pos.reshape(1, 3 * n)
    aux0 = jnp.tile(dinv, n)[None, :]
    aux1 = jnp.tile(dcel, n)[None, :]
    scal = jnp.zeros((3 * n,), jnp.float32)
    scal = scal.at[0:6].set(jnp.stack([dinv[0], dcel[0], dinv[1], dcel[1],
                                       dinv[2], dcel[2]]).astype(jnp.float32))
    onehot = jnp.tile(jnp.eye(3, dtype=jnp.float32), (1, n))
    aux = jnp.concatenate([aux0, aux1, scal[None, :], onehot], axis=0)

    kfn = functools.partial(_pair_kernel, n=n, block_i=block_i, rc2=rc2)
    adj, sft_flat, vec_flat, sod = pl.pallas_call(
        kfn,
        grid=(grid,),
        in_specs=[
            pl.BlockSpec((block_i, 3), lambda i: (i, 0)),
            pl.BlockSpec((3, n), lambda i: (0, 0)),
            pl.BlockSpec((1, 3 * n), lambda i: (0, 0)),
            pl.BlockSpec((6, 3 * n), lambda i: (0, 0)),
        ],
        out_specs=[
            pl.BlockSpec((block_i, n), lambda i: (i, 0)),
            pl.BlockSpec((block_i, 3 * n), lambda i: (i, 0)),
            pl.BlockSpec((block_i, 3 * n), lambda i: (i, 0)),
            pl.BlockSpec((block_i, n), lambda i: (i, 0)),
        ],
        out_shape=[
            jax.ShapeDtypeStruct((n, n), jnp.int32),
            jax.ShapeDtypeStruct((n, 3 * n), jnp.int32),
            jax.ShapeDtypeStruct((n, 3 * n), jnp.float32),
            jax.ShapeDtypeStruct((n, n), jnp.float32),
        ],
    )(pos, pos_t, pos_flat, aux)

    return adj, sft_flat.reshape(n, n, 3), vec_flat.reshape(n, n, 3), sod


# speedup vs baseline: 6.9904x; 6.9904x over previous
import functools

import jax
import jax.numpy as jnp
import numpy as np
from jax.experimental import pallas as pl

_RC = 6.0
_BLOCK_I = 128


def _pair_kernel(pos_blk_ref, pos_t_ref, cfg_ref,
                 adj_ref, sft_ref, vec_ref, sod_ref, *, n, block_i, rc2):
    pid = pl.program_id(0)
    rows = pid * block_i + jax.lax.broadcasted_iota(jnp.int32, (block_i, n), 0)
    cols = jax.lax.broadcasted_iota(jnp.int32, (block_i, n), 1)

    sod = jnp.zeros((block_i, n), jnp.float32)
    vs = []
    for c in range(3):
        pj = pos_t_ref[c:c + 1, :]
        pi = pos_blk_ref[:, c:c + 1]
        d = pj - pi
        f = d * cfg_ref[0, 2 * c]
        s = -jnp.round(f)
        v = d + s * cfg_ref[0, 2 * c + 1]
        sod = sod + v * v
        vs.append((v, s))

    mask = (sod < rc2) & (rows != cols)
    adj_ref[...] = mask.astype(jnp.int32)
    sod_ref[...] = jnp.where(mask, sod, 0.0)
    for c, (v, s) in enumerate(vs):
        vec_ref[c, :, :] = jnp.where(mask, v, 0.0)
        sft_ref[c, :, :] = jnp.where(mask, s, 0.0).astype(jnp.int32)


def kernel(pos, cel):
    n = pos.shape[0]
    block_i = _BLOCK_I
    grid = n // block_i
    rc2 = np.float32(_RC * _RC)

    inv_cel = jnp.linalg.inv(cel)
    dinv = jnp.diagonal(inv_cel).astype(jnp.float32)
    dcel = jnp.diagonal(cel).astype(jnp.float32)
    cfg = jnp.zeros((1, 128), jnp.float32)
    cfg = cfg.at[0, 0:6].set(jnp.stack([dinv[0], dcel[0], dinv[1], dcel[1],
                                        dinv[2], dcel[2]]))
    pos_t = pos.T

    kfn = functools.partial(_pair_kernel, n=n, block_i=block_i, rc2=rc2)
    adj, sft_p, vec_p, sod = pl.pallas_call(
        kfn,
        grid=(grid,),
        in_specs=[
            pl.BlockSpec((block_i, 3), lambda i: (i, 0)),
            pl.BlockSpec((3, n), lambda i: (0, 0)),
            pl.BlockSpec((1, 128), lambda i: (0, 0)),
        ],
        out_specs=[
            pl.BlockSpec((block_i, n), lambda i: (i, 0)),
            pl.BlockSpec((3, block_i, n), lambda i: (0, i, 0)),
            pl.BlockSpec((3, block_i, n), lambda i: (0, i, 0)),
            pl.BlockSpec((block_i, n), lambda i: (i, 0)),
        ],
        out_shape=[
            jax.ShapeDtypeStruct((n, n), jnp.int32),
            jax.ShapeDtypeStruct((3, n, n), jnp.int32),
            jax.ShapeDtypeStruct((3, n, n), jnp.float32),
            jax.ShapeDtypeStruct((n, n), jnp.float32),
        ],
    )(pos, pos_t, cfg)

    return (adj, jnp.transpose(sft_p, (1, 2, 0)),
            jnp.transpose(vec_p, (1, 2, 0)), sod)
